# baseline (speedup 1.0000x reference)
import functools

import jax
import jax.numpy as jnp
from jax import lax
from jax.experimental import pallas as pl
from jax.experimental.pallas import tpu as pltpu
from jax.experimental.pallas import tpu_sc as plsc

_NUM_QUESTION = 100000
_NUM_CONCEPT = 1000
_EMB_DIM = 128
_MAX_C = 4

_CPAD = 1008
_ZROW = _NUM_CONCEPT
_CTW = _CPAD * _EMB_DIM // 2

_NW = 32
_CH = 32
_CB = 400


def _bf16_table_body(tab_ref, out_ref):
    out_ref[...] = tab_ref[...].astype(jnp.bfloat16)


def _build_bf16_table(concept_padded):
    out = pl.pallas_call(
        _bf16_table_body,
        in_specs=[pl.BlockSpec((_CPAD, _EMB_DIM), lambda: (0, 0))],
        out_specs=pl.BlockSpec((_CPAD, _EMB_DIM), lambda: (0, 0)),
        out_shape=jax.ShapeDtypeStruct((_CPAD, _EMB_DIM), jnp.bfloat16),
    )(concept_padded)
    return lax.bitcast_convert_type(
        out.reshape(_CPAD, _EMB_DIM // 2, 2), jnp.int32).reshape(_CTW)


def _lane_gather(v, idx):
    return lax.gather(
        v, idx[:, None],
        lax.GatherDimensionNumbers(offset_dims=(), collapsed_slice_dims=(0,),
                                   start_index_map=(0,)),
        slice_sizes=(1,), mode=lax.GatherScatterMode.PROMISE_IN_BOUNDS)


def _make_sc_kernel(n_tokens):
    nt = n_tokens // _NW
    nchunk = nt // _CH
    mesh = plsc.VectorSubcoreMesh(core_axis_name="c", subcore_axis_name="s")

    @functools.partial(
        pl.kernel,
        mesh=mesh,
        compiler_params=pltpu.CompilerParams(use_tc_tiling_on_sc=False,
                                             needs_layout_passes=False),
        out_type=jax.ShapeDtypeStruct((n_tokens, 2 * _EMB_DIM), jnp.float32),
        scratch_types=[
            pltpu.VMEM((_CTW,), jnp.int32),
            pltpu.VMEM((nt,), jnp.int32),
            pltpu.VMEM((_CB, 16), jnp.int32),
            pltpu.VMEM((4 * nt + 16,), jnp.int32),
            pltpu.VMEM((4 * nt + 16,), jnp.float32),
            pltpu.VMEM((2, _CH, _EMB_DIM), jnp.float32),
            pltpu.VMEM((2, _CH, _EMB_DIM), jnp.float32),
            pltpu.SemaphoreType.DMA,
            pltpu.SemaphoreType.DMA,
            [pltpu.SemaphoreType.DMA] * 2,
            [pltpu.SemaphoreType.DMA] * 2,
            [pltpu.SemaphoreType.DMA] * 2,
        ],
    )
    def sc_kernel(qseq_hbm, combo_hbm, ctab_hbm, embq_hbm, out_hbm,
                  ctab_v, qidx_v, combo_v, eff_v, inv_v, qrows_v, fus_v,
                  csem, sem0, gsem_q, ssem_f, ssem_q):
        wid = lax.axis_index("s") * 2 + lax.axis_index("c")
        tbase = wid * nt
        lane = lax.iota(jnp.int32, 16)
        perm1 = lane ^ 1
        perm2 = lane ^ 2
        lt4 = lane < 4
        lt8 = lane < 8
        lt12 = lane < 12
        col_q = [16 * q + lane for q in range(4)]
        jfull = [jnp.full((16,), j, jnp.int32) for j in range(4)]
        zfull = jnp.full((16,), 0, jnp.int32)

        ctab_copy = pltpu.async_copy(ctab_hbm, ctab_v, csem)

        pltpu.sync_copy(qseq_hbm.at[pl.ds(tbase, nt)], qidx_v)

        for s in range(nt // _CB):
            pltpu.async_copy(
                combo_hbm.at[qidx_v.at[pl.ds(s * _CB, _CB)]], combo_v,
                sem0).wait()

            def grp_body(g, c2, s=s):
                v0 = combo_v[4 * g, :]
                v1 = combo_v[4 * g + 1, :]
                v2 = combo_v[4 * g + 2, :]
                v3 = combo_v[4 * g + 3, :]
                v = jnp.where(lt4, v0,
                              jnp.where(lt8, v1, jnp.where(lt12, v2, v3)))
                cid = v & 0xFFF
                m = lax.shift_right_logical(v, 12) & 1
                a = m + _lane_gather(m, perm1)
                cnt = a + _lane_gather(a, perm2)
                eff = jnp.where(m > 0, cid, jnp.full((16,), _ZROW, jnp.int32))
                eff_v[pl.ds(4 * _CB * s + 16 * g, 16)] = eff
                inv_v[pl.ds(4 * _CB * s + 16 * g, 16)] = (
                    1.0 / cnt.astype(jnp.float32))
                return c2

            lax.fori_loop(0, _CB // 4, grp_body, 0)

        ctab_copy.wait()

        def fire_gathers(ci, b):
            pltpu.async_copy(
                embq_hbm.at[qidx_v.at[pl.ds(ci * _CH, _CH)]],
                qrows_v.at[b], gsem_q[b])

        def drain_gathers(b):
            pltpu.make_async_copy(
                embq_hbm.at[qidx_v.at[pl.ds(0, _CH)]],
                qrows_v.at[b], gsem_q[b]).wait()

        def fire_scatters(ci, b):
            base = tbase + ci * _CH
            pltpu.async_copy(
                fus_v.at[b],
                out_hbm.at[pl.ds(base, _CH), pl.ds(0, _EMB_DIM)], ssem_f[b])
            pltpu.async_copy(
                qrows_v.at[b],
                out_hbm.at[pl.ds(base, _CH), pl.ds(_EMB_DIM, _EMB_DIM)],
                ssem_q[b])

        def drain_scatters(b):
            pltpu.make_async_copy(
                fus_v.at[b],
                out_hbm.at[pl.ds(0, _CH), pl.ds(0, _EMB_DIM)],
                ssem_f[b]).wait()
            pltpu.make_async_copy(
                qrows_v.at[b],
                out_hbm.at[pl.ds(0, _CH), pl.ds(_EMB_DIM, _EMB_DIM)],
                ssem_q[b]).wait()

        fire_gathers(0, 0)

        def outer_body(i2, carry):
            for b in (0, 1):
                ci = 2 * i2 + b
                nb = 1 - b
                if b == 0:
                    @pl.when(i2 >= 1)
                    def _():
                        drain_scatters(nb)
                    fire_gathers(ci + 1, nb)
                else:
                    drain_scatters(nb)

                    @pl.when(i2 < nchunk // 2 - 1)
                    def _():
                        fire_gathers(ci + 1, nb)

                def tok_body(t_loc, c2, b=b, ci=ci):
                    t4 = 4 * (ci * _CH + t_loc)
                    e16 = eff_v[pl.ds(t4, 16)]
                    i16 = inv_v[pl.ds(t4, 16)]
                    invb = _lane_gather(i16, zfull)
                    invpk = plsc.pack(invb, invb,
                                      format=plsc.PackFormat.INTERLEAVED)
                    rowf = jnp.full((16,), t_loc, jnp.int32)
                    wb = [lax.shift_left(_lane_gather(e16, jfull[j]), 6)
                          for j in range(4)]
                    for q in range(4):
                        acc = None
                        for j in range(4):
                            w = plsc.load_gather(ctab_v, [wb[j] + col_q[q]])
                            bfv = plsc.bitcast(w, jnp.bfloat16)
                            acc = bfv if acc is None else acc + bfv
                        acc = acc * invpk
                        f0, f1 = plsc.unpack(
                            acc, format=plsc.PackFormat.INTERLEAVED)
                        c0 = 32 * q + 2 * lane
                        plsc.store_scatter(fus_v.at[b], [rowf, c0], f0)
                        plsc.store_scatter(fus_v.at[b], [rowf, c0 + 1], f1)
                    return c2

                drain_gathers(b)
                lax.fori_loop(0, _CH, tok_body, 0)
                fire_scatters(ci, b)
            return carry

        lax.fori_loop(0, nchunk // 2, outer_body, 0)
        drain_scatters(1)

    return sc_kernel


def kernel(question_seq, embed_question, embed_concept, q2c_table, q2c_mask):
    b, l = question_seq.shape
    n = b * l
    qseq = question_seq.astype(jnp.int32).reshape(n)

    concept_padded = jnp.pad(
        embed_concept.astype(jnp.float32),
        ((0, _CPAD - _NUM_CONCEPT), (0, 0)))
    ctab = _build_bf16_table(concept_padded)

    packed = (q2c_table.astype(jnp.int32) & 0xFFF) | (
        q2c_mask.astype(jnp.int32) << 12)
    combo = jnp.tile(packed, (1, 4))

    out = _make_sc_kernel(n)(qseq, combo, ctab,
                             embed_question.astype(jnp.float32))
    return out.reshape(b, l, 2 * _EMB_DIM)


# speedup vs baseline: 1.0181x; 1.0181x over previous
import functools

import jax
import jax.numpy as jnp
from jax import lax
from jax.experimental import pallas as pl
from jax.experimental.pallas import tpu as pltpu
from jax.experimental.pallas import tpu_sc as plsc

_NUM_QUESTION = 100000
_NUM_CONCEPT = 1000
_EMB_DIM = 128
_MAX_C = 4

_CPAD = 1008
_ZROW = _NUM_CONCEPT
_CTW = _CPAD * _EMB_DIM // 2

_NW = 32
_CH = 64
_CB = 400


def _bf16_table_body(tab_ref, out_ref):
    out_ref[...] = tab_ref[...].astype(jnp.bfloat16)


def _build_bf16_table(concept_padded):
    out = pl.pallas_call(
        _bf16_table_body,
        in_specs=[pl.BlockSpec((_CPAD, _EMB_DIM), lambda: (0, 0))],
        out_specs=pl.BlockSpec((_CPAD, _EMB_DIM), lambda: (0, 0)),
        out_shape=jax.ShapeDtypeStruct((_CPAD, _EMB_DIM), jnp.bfloat16),
    )(concept_padded)
    return lax.bitcast_convert_type(
        out.reshape(_CPAD, _EMB_DIM // 2, 2), jnp.int32).reshape(_CTW)


def _lane_gather(v, idx):
    return lax.gather(
        v, idx[:, None],
        lax.GatherDimensionNumbers(offset_dims=(), collapsed_slice_dims=(0,),
                                   start_index_map=(0,)),
        slice_sizes=(1,), mode=lax.GatherScatterMode.PROMISE_IN_BOUNDS)


def _make_sc_kernel(n_tokens):
    nt = n_tokens // _NW
    nchunk = nt // _CH
    mesh = plsc.VectorSubcoreMesh(core_axis_name="c", subcore_axis_name="s")

    @functools.partial(
        pl.kernel,
        mesh=mesh,
        compiler_params=pltpu.CompilerParams(use_tc_tiling_on_sc=False,
                                             needs_layout_passes=False),
        out_type=jax.ShapeDtypeStruct((n_tokens, 2 * _EMB_DIM), jnp.float32),
        scratch_types=[
            pltpu.VMEM((_CTW,), jnp.int32),
            pltpu.VMEM((nt,), jnp.int32),
            pltpu.VMEM((_CB, 16), jnp.int32),
            pltpu.VMEM((4 * nt + 16,), jnp.int32),
            pltpu.VMEM((4 * nt + 16,), jnp.float32),
            pltpu.VMEM((2, _CH, _EMB_DIM), jnp.float32),
            pltpu.VMEM((2, _CH, _EMB_DIM), jnp.float32),
            pltpu.SemaphoreType.DMA,
            pltpu.SemaphoreType.DMA,
            [pltpu.SemaphoreType.DMA] * 2,
            [pltpu.SemaphoreType.DMA] * 2,
            [pltpu.SemaphoreType.DMA] * 2,
        ],
    )
    def sc_kernel(qseq_hbm, combo_hbm, ctab_hbm, embq_hbm, out_hbm,
                  ctab_v, qidx_v, combo_v, eff_v, inv_v, qrows_v, fus_v,
                  csem, sem0, gsem_q, ssem_f, ssem_q):
        wid = lax.axis_index("s") * 2 + lax.axis_index("c")
        tbase = wid * nt
        lane = lax.iota(jnp.int32, 16)
        perm1 = lane ^ 1
        perm2 = lane ^ 2
        lt4 = lane < 4
        lt8 = lane < 8
        lt12 = lane < 12
        col_q = [16 * q + lane for q in range(4)]
        c0_q = [32 * q + 2 * lane for q in range(4)]
        c1_q = [32 * q + 2 * lane + 1 for q in range(4)]
        jfull = [jnp.full((16,), j, jnp.int32) for j in range(4)]
        zfull = jnp.full((16,), 0, jnp.int32)

        ctab_copy = pltpu.async_copy(ctab_hbm, ctab_v, csem)

        pltpu.sync_copy(qseq_hbm.at[pl.ds(tbase, nt)], qidx_v)

        for s in range(nt // _CB):
            pltpu.async_copy(
                combo_hbm.at[qidx_v.at[pl.ds(s * _CB, _CB)]], combo_v,
                sem0).wait()

            def grp_body(g, c2, s=s):
                v0 = combo_v[4 * g, :]
                v1 = combo_v[4 * g + 1, :]
                v2 = combo_v[4 * g + 2, :]
                v3 = combo_v[4 * g + 3, :]
                v = jnp.where(lt4, v0,
                              jnp.where(lt8, v1, jnp.where(lt12, v2, v3)))
                cid = v & 0xFFF
                m = lax.shift_right_logical(v, 12) & 1
                a = m + _lane_gather(m, perm1)
                cnt = a + _lane_gather(a, perm2)
                eff = jnp.where(m > 0, cid, jnp.full((16,), _ZROW, jnp.int32))
                eff_v[pl.ds(4 * _CB * s + 16 * g, 16)] = lax.shift_left(eff, 6)
                inv_v[pl.ds(4 * _CB * s + 16 * g, 16)] = (
                    1.0 / cnt.astype(jnp.float32))
                return c2

            lax.fori_loop(0, _CB // 4, grp_body, 0)

        ctab_copy.wait()

        def fire_gathers(ci, b):
            pltpu.async_copy(
                embq_hbm.at[qidx_v.at[pl.ds(ci * _CH, _CH)]],
                qrows_v.at[b], gsem_q[b])

        def drain_gathers(b):
            pltpu.make_async_copy(
                embq_hbm.at[qidx_v.at[pl.ds(0, _CH)]],
                qrows_v.at[b], gsem_q[b]).wait()

        def fire_scatters(ci, b):
            base = tbase + ci * _CH
            pltpu.async_copy(
                fus_v.at[b],
                out_hbm.at[pl.ds(base, _CH), pl.ds(0, _EMB_DIM)], ssem_f[b])
            pltpu.async_copy(
                qrows_v.at[b],
                out_hbm.at[pl.ds(base, _CH), pl.ds(_EMB_DIM, _EMB_DIM)],
                ssem_q[b])

        def drain_scatters(b):
            pltpu.make_async_copy(
                fus_v.at[b],
                out_hbm.at[pl.ds(0, _CH), pl.ds(0, _EMB_DIM)],
                ssem_f[b]).wait()
            pltpu.make_async_copy(
                qrows_v.at[b],
                out_hbm.at[pl.ds(0, _CH), pl.ds(_EMB_DIM, _EMB_DIM)],
                ssem_q[b]).wait()

        fire_gathers(0, 0)

        def outer_body(i2, carry):
            for b in (0, 1):
                ci = 2 * i2 + b
                nb = 1 - b
                if b == 0:
                    @pl.when(i2 >= 1)
                    def _():
                        drain_scatters(nb)
                    fire_gathers(ci + 1, nb)
                else:
                    drain_scatters(nb)

                    @pl.when(i2 < nchunk // 2 - 1)
                    def _():
                        fire_gathers(ci + 1, nb)

                def tok_body(t_loc, c2, b=b, ci=ci):
                    t4 = 4 * (ci * _CH + t_loc)
                    e16 = eff_v[pl.ds(t4, 16)]
                    i16 = inv_v[pl.ds(t4, 16)]
                    invb = _lane_gather(i16, zfull)
                    invpk = plsc.pack(invb, invb,
                                      format=plsc.PackFormat.INTERLEAVED)
                    rowf = jnp.full((16,), t_loc, jnp.int32)
                    wb = [_lane_gather(e16, jfull[j]) for j in range(4)]
                    for q in range(4):
                        acc = None
                        for j in range(4):
                            w = plsc.load_gather(ctab_v, [wb[j] + col_q[q]])
                            bfv = plsc.bitcast(w, jnp.bfloat16)
                            acc = bfv if acc is None else acc + bfv
                        acc = acc * invpk
                        f0, f1 = plsc.unpack(
                            acc, format=plsc.PackFormat.INTERLEAVED)
                        plsc.store_scatter(fus_v.at[b], [rowf, c0_q[q]], f0)
                        plsc.store_scatter(fus_v.at[b], [rowf, c1_q[q]], f1)
                    return c2

                drain_gathers(b)
                lax.fori_loop(0, _CH, tok_body, 0)
                fire_scatters(ci, b)
            return carry

        lax.fori_loop(0, nchunk // 2, outer_body, 0)
        drain_scatters(1)

    return sc_kernel


def kernel(question_seq, embed_question, embed_concept, q2c_table, q2c_mask):
    b, l = question_seq.shape
    n = b * l
    qseq = question_seq.astype(jnp.int32).reshape(n)

    concept_padded = jnp.pad(
        embed_concept.astype(jnp.float32),
        ((0, _CPAD - _NUM_CONCEPT), (0, 0)))
    ctab = _build_bf16_table(concept_padded)

    packed = (q2c_table.astype(jnp.int32) & 0xFFF) | (
        q2c_mask.astype(jnp.int32) << 12)
    combo = jnp.tile(packed, (1, 4))

    out = _make_sc_kernel(n)(qseq, combo, ctab,
                             embed_question.astype(jnp.float32))
    return out.reshape(b, l, 2 * _EMB_DIM)
